# baseline (device time: 63727 ns/iter reference)
import jax
import jax.numpy as jnp
from jax import lax
from jax.experimental import pallas as pl
from jax.experimental.pallas import tpu as pltpu

N_DEV = 32
B, SQ, DM = 2, 256, 512
DH = 64
H_PER = 4
ROWS = B * SQ
CHUNK = ROWS // N_DEV

import os
DO_RS = os.environ.get("KERNEL_NO_RS") != "1"
DO_AG = os.environ.get("KERNEL_NO_AG") != "1"


def kernel(x, Wq, K_ext, V_ext, Wo):
    K_ext = K_ext.reshape(B, SQ, 128 * DH)
    V_ext = V_ext.reshape(B, SQ, 128 * DH)

    def body(x_ref, wq_ref, k_hbm, v_hbm, wo_ref, out_ref,
             k_ref, v_ref, acc_ref, stage_ref,
             kv_sems, rs_send_sems, rs_recv_sems, ag_send_sems, ag_recv_sems):
        me = lax.axis_index("i")

        kcopy = pltpu.make_async_copy(k_hbm, k_ref, kv_sems.at[0])
        vcopy = pltpu.make_async_copy(v_hbm, v_ref, kv_sems.at[1])
        kcopy.start()
        vcopy.start()

        stage_ref[pl.ds(me, 1)] = jnp.zeros((1, CHUNK, DM), jnp.float32)

        x2 = x_ref[...].reshape(ROWS, DM)
        q = jnp.dot(x2, wq_ref[...], preferred_element_type=jnp.float32)
        q4 = q.reshape(B, SQ, H_PER, DH)

        qb = lax.broadcasted_iota(jnp.int32, (SQ, SQ), 0) // 64
        kb = lax.broadcasted_iota(jnp.int32, (SQ, SQ), 1) // 64
        mask = (qb == kb) | ((kb % 4) == (qb % 4))

        kcopy.wait()
        vcopy.wait()

        for b in range(B):
            ctxs = []
            for h in range(H_PER):
                h2, hh = divmod(h, 2)
                kp = k_ref[b, :, pl.ds(me * 256 + h2 * 128, 128)]
                vp = v_ref[b, :, pl.ds(me * 256 + h2 * 128, 128)]
                qh = q4[b, :, h, :]
                kh = kp[:, hh * DH:(hh + 1) * DH]
                vh = vp[:, hh * DH:(hh + 1) * DH]
                s = lax.dot_general(
                    qh, kh, (((1,), (1,)), ((), ())),
                    preferred_element_type=jnp.float32) * 0.125
                s = jnp.where(mask, s, -1e9)
                w = jnp.exp(s - jnp.max(s, axis=-1, keepdims=True))
                w = w / jnp.sum(w, axis=-1, keepdims=True)
                ctxs.append(jnp.dot(w, vh, preferred_element_type=jnp.float32))
            ctx_flat = jnp.concatenate(ctxs, axis=1)
            pb = jnp.dot(ctx_flat, wo_ref[...],
                         preferred_element_type=jnp.float32)
            acc_ref[pl.ds(b * 16, 16)] = pb.reshape(16, CHUNK, DM)

        for off in range(1, N_DEV if DO_RS else 1):
            t = lax.rem(me + off, N_DEV)
            pltpu.make_async_remote_copy(
                src_ref=acc_ref.at[t],
                dst_ref=stage_ref.at[me],
                send_sem=rs_send_sems.at[t],
                recv_sem=rs_recv_sems.at[me],
                device_id=(t,),
                device_id_type=pl.DeviceIdType.MESH,
            ).start()

        for off in range(1, N_DEV if DO_RS else 1):
            j = lax.rem(me + off, N_DEV)
            pltpu.make_async_remote_copy(
                src_ref=acc_ref.at[j],
                dst_ref=stage_ref.at[j],
                send_sem=rs_send_sems.at[j],
                recv_sem=rs_recv_sems.at[j],
                device_id=(j,),
                device_id_type=pl.DeviceIdType.MESH,
            ).wait_recv()
        red = (acc_ref[pl.ds(me, 1)]
               + jnp.sum(stage_ref[...], axis=0, keepdims=True))
        acc_ref[pl.ds(me, 1)] = red

        for off in range(1, N_DEV if DO_AG else 1):
            t = lax.rem(me + off, N_DEV)
            pltpu.make_async_remote_copy(
                src_ref=acc_ref.at[me],
                dst_ref=acc_ref.at[me],
                send_sem=ag_send_sems.at[t],
                recv_sem=ag_recv_sems.at[me],
                device_id=(t,),
                device_id_type=pl.DeviceIdType.MESH,
            ).start()

        for off in range(1, N_DEV if DO_AG else 1):
            j = lax.rem(me + off, N_DEV)
            pltpu.make_async_remote_copy(
                src_ref=acc_ref.at[j],
                dst_ref=acc_ref.at[j],
                send_sem=ag_send_sems.at[j],
                recv_sem=ag_recv_sems.at[j],
                device_id=(j,),
                device_id_type=pl.DeviceIdType.MESH,
            ).wait_recv()

        for off in range(1, N_DEV):
            t = lax.rem(me + off, N_DEV)
            if DO_RS:
                pltpu.make_async_remote_copy(
                    src_ref=acc_ref.at[t],
                    dst_ref=stage_ref.at[me],
                    send_sem=rs_send_sems.at[t],
                    recv_sem=rs_recv_sems.at[me],
                    device_id=(t,),
                    device_id_type=pl.DeviceIdType.MESH,
                ).wait_send()
            if DO_AG:
                pltpu.make_async_remote_copy(
                    src_ref=acc_ref.at[me],
                    dst_ref=acc_ref.at[me],
                    send_sem=ag_send_sems.at[t],
                    recv_sem=ag_recv_sems.at[me],
                    device_id=(t,),
                    device_id_type=pl.DeviceIdType.MESH,
                ).wait_send()

        out_ref[...] = acc_ref[...].reshape(B, SQ, DM)

    return pl.pallas_call(
        body,
        out_shape=jax.ShapeDtypeStruct((B, SQ, DM), jnp.float32),
        in_specs=[
            pl.BlockSpec(memory_space=pltpu.VMEM),
            pl.BlockSpec(memory_space=pltpu.VMEM),
            pl.BlockSpec(memory_space=pl.ANY),
            pl.BlockSpec(memory_space=pl.ANY),
            pl.BlockSpec(memory_space=pltpu.VMEM),
        ],
        out_specs=pl.BlockSpec(memory_space=pltpu.VMEM),
        scratch_shapes=[
            pltpu.VMEM((B, SQ, 128 * DH), jnp.float32),
            pltpu.VMEM((B, SQ, 128 * DH), jnp.float32),
            pltpu.VMEM((N_DEV, CHUNK, DM), jnp.float32),
            pltpu.VMEM((N_DEV, CHUNK, DM), jnp.float32),
            pltpu.SemaphoreType.DMA((2,)),
            pltpu.SemaphoreType.DMA((N_DEV,)),
            pltpu.SemaphoreType.DMA((N_DEV,)),
            pltpu.SemaphoreType.DMA((N_DEV,)),
            pltpu.SemaphoreType.DMA((N_DEV,)),
        ],
        compiler_params=pltpu.CompilerParams(
            vmem_limit_bytes=50 * 1024 * 1024,
        ),
    )(x, Wq, K_ext, V_ext, Wo)


# device time: 51452 ns/iter; 1.2386x vs baseline; 1.2386x over previous
import jax
import jax.numpy as jnp
from jax import lax
from jax.experimental import pallas as pl
from jax.experimental.pallas import tpu as pltpu

N_DEV = 32
B, SQ, DM = 2, 256, 512
DH = 64
H_PER = 4
ROWS = B * SQ
CHUNK = ROWS // N_DEV

import os
DO_RS = os.environ.get("KERNEL_NO_RS") != "1"
DO_AG = os.environ.get("KERNEL_NO_AG") != "1"


def kernel(x, Wq, K_ext, V_ext, Wo):
    def body(x_ref, wq_ref, k_hbm, v_hbm, wo_ref, out_ref,
             k_ref, v_ref, acc_ref, stage_ref,
             kv_sems, rs_send_sems, rs_recv_sems, ag_send_sems, ag_recv_sems):
        me = lax.axis_index("i")

        NSPLIT = 8
        SROWS = SQ // NSPLIT
        kv_copies = []
        for b in range(B):
            for c in range(NSPLIT):
                i = b * NSPLIT + c
                kv_copies.append(pltpu.make_async_copy(
                    k_hbm.at[b, pl.ds(c * SROWS, SROWS),
                             pl.ds(me * H_PER, H_PER), :],
                    k_ref.at[b, pl.ds(c * SROWS, SROWS)],
                    kv_sems.at[i]))
                kv_copies.append(pltpu.make_async_copy(
                    v_hbm.at[b, pl.ds(c * SROWS, SROWS),
                             pl.ds(me * H_PER, H_PER), :],
                    v_ref.at[b, pl.ds(c * SROWS, SROWS)],
                    kv_sems.at[B * NSPLIT + i]))
        for cp in kv_copies:
            cp.start()

        stage_ref[pl.ds(me, 1)] = jnp.zeros((1, CHUNK, DM), jnp.float32)

        x2 = x_ref[...].reshape(ROWS, DM)
        q = jnp.dot(x2, wq_ref[...], preferred_element_type=jnp.float32)
        q4 = q.reshape(B, SQ, H_PER, DH)

        qb = lax.broadcasted_iota(jnp.int32, (SQ, SQ), 0) // 64
        kb = lax.broadcasted_iota(jnp.int32, (SQ, SQ), 1) // 64
        mask = (qb == kb) | ((kb % 4) == (qb % 4))

        for cp in kv_copies:
            cp.wait()

        for b in range(B):
            ctxs = []
            for h in range(H_PER):
                qh = q4[b, :, h, :]
                kh = k_ref[b, :, h, :]
                vh = v_ref[b, :, h, :]
                s = lax.dot_general(
                    qh, kh, (((1,), (1,)), ((), ())),
                    preferred_element_type=jnp.float32) * 0.125
                s = jnp.where(mask, s, -1e9)
                w = jnp.exp(s - jnp.max(s, axis=-1, keepdims=True))
                w = w / jnp.sum(w, axis=-1, keepdims=True)
                ctxs.append(jnp.dot(w, vh, preferred_element_type=jnp.float32))
            ctx_flat = jnp.concatenate(ctxs, axis=1)
            pb = jnp.dot(ctx_flat, wo_ref[...],
                         preferred_element_type=jnp.float32)
            acc_ref[pl.ds(b * 16, 16)] = pb.reshape(16, CHUNK, DM)

        for off in range(1, N_DEV if DO_RS else 1):
            t = lax.rem(me + off, N_DEV)
            pltpu.make_async_remote_copy(
                src_ref=acc_ref.at[t],
                dst_ref=stage_ref.at[me],
                send_sem=rs_send_sems.at[t],
                recv_sem=rs_recv_sems.at[me],
                device_id=(t,),
                device_id_type=pl.DeviceIdType.MESH,
            ).start()

        for off in range(1, N_DEV if DO_RS else 1):
            j = lax.rem(me + off, N_DEV)
            pltpu.make_async_remote_copy(
                src_ref=acc_ref.at[j],
                dst_ref=stage_ref.at[j],
                send_sem=rs_send_sems.at[j],
                recv_sem=rs_recv_sems.at[j],
                device_id=(j,),
                device_id_type=pl.DeviceIdType.MESH,
            ).wait_recv()
        red = (acc_ref[pl.ds(me, 1)]
               + jnp.sum(stage_ref[...], axis=0, keepdims=True))
        acc_ref[pl.ds(me, 1)] = red

        for off in range(1, N_DEV if DO_AG else 1):
            t = lax.rem(me + off, N_DEV)
            pltpu.make_async_remote_copy(
                src_ref=acc_ref.at[me],
                dst_ref=acc_ref.at[me],
                send_sem=ag_send_sems.at[t],
                recv_sem=ag_recv_sems.at[me],
                device_id=(t,),
                device_id_type=pl.DeviceIdType.MESH,
            ).start()

        for off in range(1, N_DEV if DO_AG else 1):
            j = lax.rem(me + off, N_DEV)
            pltpu.make_async_remote_copy(
                src_ref=acc_ref.at[j],
                dst_ref=acc_ref.at[j],
                send_sem=ag_send_sems.at[j],
                recv_sem=ag_recv_sems.at[j],
                device_id=(j,),
                device_id_type=pl.DeviceIdType.MESH,
            ).wait_recv()

        for off in range(1, N_DEV):
            t = lax.rem(me + off, N_DEV)
            if DO_RS:
                pltpu.make_async_remote_copy(
                    src_ref=acc_ref.at[t],
                    dst_ref=stage_ref.at[me],
                    send_sem=rs_send_sems.at[t],
                    recv_sem=rs_recv_sems.at[me],
                    device_id=(t,),
                    device_id_type=pl.DeviceIdType.MESH,
                ).wait_send()
            if DO_AG:
                pltpu.make_async_remote_copy(
                    src_ref=acc_ref.at[me],
                    dst_ref=acc_ref.at[me],
                    send_sem=ag_send_sems.at[t],
                    recv_sem=ag_recv_sems.at[me],
                    device_id=(t,),
                    device_id_type=pl.DeviceIdType.MESH,
                ).wait_send()

        out_ref[...] = acc_ref[...].reshape(B, SQ, DM)

    return pl.pallas_call(
        body,
        out_shape=jax.ShapeDtypeStruct((B, SQ, DM), jnp.float32),
        in_specs=[
            pl.BlockSpec(memory_space=pltpu.VMEM),
            pl.BlockSpec(memory_space=pltpu.VMEM),
            pl.BlockSpec(memory_space=pl.ANY),
            pl.BlockSpec(memory_space=pl.ANY),
            pl.BlockSpec(memory_space=pltpu.VMEM),
        ],
        out_specs=pl.BlockSpec(memory_space=pltpu.VMEM),
        scratch_shapes=[
            pltpu.VMEM((B, SQ, H_PER, DH), jnp.float32),
            pltpu.VMEM((B, SQ, H_PER, DH), jnp.float32),
            pltpu.VMEM((N_DEV, CHUNK, DM), jnp.float32),
            pltpu.VMEM((N_DEV, CHUNK, DM), jnp.float32),
            pltpu.SemaphoreType.DMA((32,)),
            pltpu.SemaphoreType.DMA((N_DEV,)),
            pltpu.SemaphoreType.DMA((N_DEV,)),
            pltpu.SemaphoreType.DMA((N_DEV,)),
            pltpu.SemaphoreType.DMA((N_DEV,)),
        ],
        compiler_params=pltpu.CompilerParams(
            vmem_limit_bytes=50 * 1024 * 1024,
        ),
    )(x, Wq, K_ext, V_ext, Wo)
